# baseline (speedup 1.0000x reference)
import math

import jax
import jax.numpy as jnp
from jax import lax
from jax.experimental import pallas as pl


def _rms(x, eps=1e-6):
    var = jnp.mean(x * x, axis=-1, keepdims=True)
    return x * lax.rsqrt(var + eps)


def _core(p_ref, t_ref, o_ref):
    d = p_ref.shape[-1]
    scale = 1.0 / math.sqrt(d)
    for k in range(p_ref.shape[0]):
        p = p_ref[k]
        t = t_ref[k]
        pn = _rms(p)
        tn = _rms(t)
        logits = lax.dot_general(
            pn, tn, (((1,), (1,)), ((), ())),
            preferred_element_type=jnp.float32) * scale
        attn = jax.nn.softmax(logits, axis=-1)
        q = lax.dot_general(
            attn, tn, (((1,), (0,)), ((), ())),
            preferred_element_type=jnp.float32)
        qn = _rms(q)
        score = lax.dot_general(
            qn, pn, (((1,), (1,)), ((), ())),
            preferred_element_type=jnp.float32) * scale
        m = jnp.max(score, axis=-1, keepdims=True)
        o_ref[k] = jnp.where(score == m, 1.0, 0.0).astype(jnp.float32)


def kernel(patches, task_tokens):
    b, n, d = patches.shape
    t = task_tokens.shape[1]
    bb = 2
    return pl.pallas_call(
        _core,
        grid=(b // bb,),
        in_specs=[
            pl.BlockSpec((bb, n, d), lambda i: (i, 0, 0)),
            pl.BlockSpec((bb, t, d), lambda i: (i, 0, 0)),
        ],
        out_specs=pl.BlockSpec((bb, n, n), lambda i: (i, 0, 0)),
        out_shape=jax.ShapeDtypeStruct((b, n, n), jnp.float32),
    )(patches, task_tokens)
